# initial kernel scaffold (unmeasured)
import jax
import jax.numpy as jnp
from jax import lax
from jax.experimental import pallas as pl
from jax.experimental.pallas import tpu as pltpu


def kernel(x, dy):
    k_per, d = x.shape
    _, f = dy.shape
    d_half = d // 2
    f_half = f // 2

    def body(x_ref, dy_ref, out_ref, sendx, recvx, sendy, recvy, sems):
        mx = lax.axis_index("x")
        my = lax.axis_index("y")

        barrier = pltpu.get_barrier_semaphore()
        pl.semaphore_signal(barrier, inc=1, device_id=(1 - mx, my),
                            device_id_type=pl.DeviceIdType.MESH)
        pl.semaphore_signal(barrier, inc=1, device_id=(mx, 1 - my),
                            device_id_type=pl.DeviceIdType.MESH)
        pl.semaphore_wait(barrier, 2)

        xv = x_ref[...].astype(jnp.bfloat16)
        dyv = dy_ref[:, pl.ds(my * f_half, f_half)].astype(jnp.bfloat16)
        p = lax.dot_general(xv, dyv, (((0,), (0,)), ((), ())),
                            preferred_element_type=jnp.float32)

        sendx[...] = lax.dynamic_slice(
            p, ((1 - mx) * d_half, 0), (d_half, f_half)).astype(jnp.bfloat16)
        rdma_x = pltpu.make_async_remote_copy(
            src_ref=sendx, dst_ref=recvx,
            send_sem=sems.at[0], recv_sem=sems.at[1],
            device_id=(1 - mx, my), device_id_type=pl.DeviceIdType.MESH)
        rdma_x.start()
        rdma_x.wait()

        r = (lax.dynamic_slice(p, (mx * d_half, 0), (d_half, f_half))
             + recvx[...].astype(jnp.float32))

        sendy[...] = r.astype(jnp.bfloat16)
        rdma_y = pltpu.make_async_remote_copy(
            src_ref=sendy, dst_ref=recvy,
            send_sem=sems.at[2], recv_sem=sems.at[3],
            device_id=(mx, 1 - my), device_id_type=pl.DeviceIdType.MESH)
        rdma_y.start()
        rdma_y.wait()

        out_ref[:, pl.ds(my * f_half, f_half)] = r
        out_ref[:, pl.ds((1 - my) * f_half, f_half)] = (
            recvy[...].astype(jnp.float32))

    return pl.pallas_call(
        body,
        out_shape=jax.ShapeDtypeStruct((d_half, f), jnp.float32),
        in_specs=[pl.BlockSpec(memory_space=pltpu.VMEM),
                  pl.BlockSpec(memory_space=pltpu.VMEM)],
        out_specs=pl.BlockSpec(memory_space=pltpu.VMEM),
        scratch_shapes=[
            pltpu.VMEM((d_half, f_half), jnp.bfloat16),
            pltpu.VMEM((d_half, f_half), jnp.bfloat16),
            pltpu.VMEM((d_half, f_half), jnp.bfloat16),
            pltpu.VMEM((d_half, f_half), jnp.bfloat16),
            pltpu.SemaphoreType.DMA((4,)),
        ],
        compiler_params=pltpu.CompilerParams(collective_id=0),
    )(x, dy)


# baseline (device time: 73730 ns/iter reference)
import jax
import jax.numpy as jnp
from jax import lax
from jax.experimental import pallas as pl
from jax.experimental.pallas import tpu as pltpu


def kernel(x, dy):
    k_per, d = x.shape
    _, f = dy.shape
    d_half = d // 2
    f_half = f // 2

    def body(x_ref, dy_ref, out_ref, p_ref, sendx, recvx, sendy, recvy, sems):
        mx = lax.axis_index("x")
        my = lax.axis_index("y")

        barrier = pltpu.get_barrier_semaphore()
        pl.semaphore_signal(barrier, inc=1, device_id=(1 - mx, my),
                            device_id_type=pl.DeviceIdType.MESH)
        pl.semaphore_signal(barrier, inc=1, device_id=(mx, 1 - my),
                            device_id_type=pl.DeviceIdType.MESH)
        pl.semaphore_wait(barrier, 2)

        xv = x_ref[...].astype(jnp.bfloat16)
        dyv = dy_ref[:, pl.ds(my * f_half, f_half)].astype(jnp.bfloat16)
        p_ref[...] = lax.dot_general(xv, dyv, (((0,), (0,)), ((), ())),
                                     preferred_element_type=jnp.float32)

        sendx[...] = p_ref[pl.ds((1 - mx) * d_half, d_half), :].astype(
            jnp.bfloat16)
        rdma_x = pltpu.make_async_remote_copy(
            src_ref=sendx, dst_ref=recvx,
            send_sem=sems.at[0], recv_sem=sems.at[1],
            device_id=(1 - mx, my), device_id_type=pl.DeviceIdType.MESH)
        rdma_x.start()
        rdma_x.wait()

        r = (p_ref[pl.ds(mx * d_half, d_half), :]
             + recvx[...].astype(jnp.float32))

        sendy[...] = r.astype(jnp.bfloat16)
        rdma_y = pltpu.make_async_remote_copy(
            src_ref=sendy, dst_ref=recvy,
            send_sem=sems.at[2], recv_sem=sems.at[3],
            device_id=(mx, 1 - my), device_id_type=pl.DeviceIdType.MESH)
        rdma_y.start()
        rdma_y.wait()

        out_ref[:, pl.ds(my * f_half, f_half)] = r
        out_ref[:, pl.ds((1 - my) * f_half, f_half)] = (
            recvy[...].astype(jnp.float32))

    return pl.pallas_call(
        body,
        out_shape=jax.ShapeDtypeStruct((d_half, f), jnp.float32),
        in_specs=[pl.BlockSpec(memory_space=pltpu.VMEM),
                  pl.BlockSpec(memory_space=pltpu.VMEM)],
        out_specs=pl.BlockSpec(memory_space=pltpu.VMEM),
        scratch_shapes=[
            pltpu.VMEM((d, f_half), jnp.float32),
            pltpu.VMEM((d_half, f_half), jnp.bfloat16),
            pltpu.VMEM((d_half, f_half), jnp.bfloat16),
            pltpu.VMEM((d_half, f_half), jnp.bfloat16),
            pltpu.VMEM((d_half, f_half), jnp.bfloat16),
            pltpu.SemaphoreType.DMA((4,)),
        ],
        compiler_params=pltpu.CompilerParams(
            collective_id=0, vmem_limit_bytes=100 * 1024 * 1024),
    )(x, dy)


# device time: 51817 ns/iter; 1.4229x vs baseline; 1.4229x over previous
import jax
import jax.numpy as jnp
from jax import lax
from jax.experimental import pallas as pl
from jax.experimental.pallas import tpu as pltpu

NC = 8


def kernel(x, dy):
    k_per, d = x.shape
    _, f = dy.shape
    d_half = d // 2
    f_half = f // 2
    fc = f_half // NC

    def body(x_ref, dy_ref, out_ref, sendx, recvx, sendy, recvy,
             sx_sems, rx_sems, sy_sems, ry_sems):
        mx = lax.axis_index("x")
        my = lax.axis_index("y")
        dn = (((0,), (0,)), ((), ()))

        barrier = pltpu.get_barrier_semaphore()
        pl.semaphore_signal(barrier, inc=1, device_id=(1 - mx, my),
                            device_id_type=pl.DeviceIdType.MESH)
        pl.semaphore_signal(barrier, inc=1, device_id=(mx, 1 - my),
                            device_id_type=pl.DeviceIdType.MESH)
        pl.semaphore_wait(barrier, 2)

        xs_send = x_ref[:, pl.ds((1 - mx) * d_half, d_half)].astype(
            jnp.bfloat16)
        xs_keep = x_ref[:, pl.ds(mx * d_half, d_half)].astype(jnp.bfloat16)

        rx = [None] * NC
        ry = [None] * NC

        def compute_and_send_x(c):
            dyc = dy_ref[:, pl.ds(my * f_half + c * fc, fc)].astype(
                jnp.bfloat16)
            p_send = lax.dot_general(xs_send, dyc, dn,
                                     preferred_element_type=jnp.float32)
            sendx[c, ...] = p_send.astype(jnp.bfloat16)
            r = pltpu.make_async_remote_copy(
                src_ref=sendx.at[c], dst_ref=recvx.at[c],
                send_sem=sx_sems.at[c], recv_sem=rx_sems.at[c],
                device_id=(1 - mx, my), device_id_type=pl.DeviceIdType.MESH)
            r.start()
            rx[c] = r
            p_keep = lax.dot_general(xs_keep, dyc, dn,
                                     preferred_element_type=jnp.float32)
            out_ref[:, pl.ds(my * f_half + c * fc, fc)] = p_keep

        def reduce_and_send_y(c):
            rx[c].wait_recv()
            cols = pl.ds(my * f_half + c * fc, fc)
            red = out_ref[:, cols] + recvx[c, ...].astype(jnp.float32)
            out_ref[:, cols] = red
            sendy[c, ...] = red.astype(jnp.bfloat16)
            r = pltpu.make_async_remote_copy(
                src_ref=sendy.at[c], dst_ref=recvy.at[c],
                send_sem=sy_sems.at[c], recv_sem=ry_sems.at[c],
                device_id=(mx, 1 - my), device_id_type=pl.DeviceIdType.MESH)
            r.start()
            ry[c] = r

        def recv_y(c):
            ry[c].wait_recv()
            out_ref[:, pl.ds((1 - my) * f_half + c * fc, fc)] = (
                recvy[c, ...].astype(jnp.float32))

        for c in range(NC):
            compute_and_send_x(c)
            if c >= 1:
                reduce_and_send_y(c - 1)
            if c >= 2:
                recv_y(c - 2)
        reduce_and_send_y(NC - 1)
        recv_y(NC - 2)
        recv_y(NC - 1)
        for c in range(NC):
            rx[c].wait_send()
            ry[c].wait_send()

    return pl.pallas_call(
        body,
        out_shape=jax.ShapeDtypeStruct((d_half, f), jnp.float32),
        in_specs=[pl.BlockSpec(memory_space=pltpu.VMEM),
                  pl.BlockSpec(memory_space=pltpu.VMEM)],
        out_specs=pl.BlockSpec(memory_space=pltpu.VMEM),
        scratch_shapes=[
            pltpu.VMEM((NC, d_half, fc), jnp.bfloat16),
            pltpu.VMEM((NC, d_half, fc), jnp.bfloat16),
            pltpu.VMEM((NC, d_half, fc), jnp.bfloat16),
            pltpu.VMEM((NC, d_half, fc), jnp.bfloat16),
            pltpu.SemaphoreType.DMA((NC,)),
            pltpu.SemaphoreType.DMA((NC,)),
            pltpu.SemaphoreType.DMA((NC,)),
            pltpu.SemaphoreType.DMA((NC,)),
        ],
        compiler_params=pltpu.CompilerParams(
            collective_id=0, vmem_limit_bytes=100 * 1024 * 1024),
    )(x, dy)


# device time: 50672 ns/iter; 1.4550x vs baseline; 1.0226x over previous
import jax
import jax.numpy as jnp
from jax import lax
from jax.experimental import pallas as pl
from jax.experimental.pallas import tpu as pltpu

CHUNKS = (256, 256, 256, 256, 256, 256, 256, 256)
OFFS = tuple(sum(CHUNKS[:i]) for i in range(len(CHUNKS)))
NC = len(CHUNKS)


def kernel(x, dy):
    k_per, d = x.shape
    _, f = dy.shape
    d_half = d // 2
    f_half = f // 2
    assert sum(CHUNKS) == f_half

    def body(x_ref, dy_ref, out_ref, *scratch):
        sendx = scratch[0:NC]
        recvx = scratch[NC:2 * NC]
        sendy = scratch[2 * NC:3 * NC]
        recvy = scratch[3 * NC:4 * NC]
        sx_sems, rx_sems, sy_sems, ry_sems = scratch[4 * NC:]

        mx = lax.axis_index("x")
        my = lax.axis_index("y")
        dn = (((0,), (0,)), ((), ()))

        barrier = pltpu.get_barrier_semaphore()
        pl.semaphore_signal(barrier, inc=1, device_id=(1 - mx, my),
                            device_id_type=pl.DeviceIdType.MESH)
        pl.semaphore_signal(barrier, inc=1, device_id=(mx, 1 - my),
                            device_id_type=pl.DeviceIdType.MESH)
        pl.semaphore_wait(barrier, 2)

        xs_send = x_ref[:, pl.ds((1 - mx) * d_half, d_half)].astype(
            jnp.bfloat16)
        xs_keep = x_ref[:, pl.ds(mx * d_half, d_half)].astype(jnp.bfloat16)

        rx = [None] * NC
        ry = [None] * NC
        dycs = [None] * NC

        for c in range(NC):
            dycs[c] = dy_ref[:, pl.ds(my * f_half + OFFS[c],
                                      CHUNKS[c])].astype(jnp.bfloat16)
            p_send = lax.dot_general(xs_send, dycs[c], dn,
                                     preferred_element_type=jnp.float32)
            sendx[c][...] = p_send.astype(jnp.bfloat16)
            r = pltpu.make_async_remote_copy(
                src_ref=sendx[c], dst_ref=recvx[c],
                send_sem=sx_sems.at[c], recv_sem=rx_sems.at[c],
                device_id=(1 - mx, my), device_id_type=pl.DeviceIdType.MESH)
            r.start()
            rx[c] = r

        def recv_y(c):
            ry[c].wait_recv()
            out_ref[:, pl.ds((1 - my) * f_half + OFFS[c], CHUNKS[c])] = (
                recvy[c][...].astype(jnp.float32))

        for c in range(NC):
            pk = lax.dot_general(xs_keep, dycs[c], dn,
                                 preferred_element_type=jnp.float32)
            rx[c].wait_recv()
            red = pk + recvx[c][...].astype(jnp.float32)
            out_ref[:, pl.ds(my * f_half + OFFS[c], CHUNKS[c])] = red
            sendy[c][...] = red.astype(jnp.bfloat16)
            r = pltpu.make_async_remote_copy(
                src_ref=sendy[c], dst_ref=recvy[c],
                send_sem=sy_sems.at[c], recv_sem=ry_sems.at[c],
                device_id=(mx, 1 - my), device_id_type=pl.DeviceIdType.MESH)
            r.start()
            ry[c] = r
            if c >= 2:
                recv_y(c - 2)

        recv_y(NC - 2)
        recv_y(NC - 1)
        for c in range(NC):
            rx[c].wait_send()
            ry[c].wait_send()

    comm_bufs = [pltpu.VMEM((d_half, CHUNKS[c]), jnp.bfloat16)
                 for _ in range(4) for c in range(NC)]
    return pl.pallas_call(
        body,
        out_shape=jax.ShapeDtypeStruct((d_half, f), jnp.float32),
        in_specs=[pl.BlockSpec(memory_space=pltpu.VMEM),
                  pl.BlockSpec(memory_space=pltpu.VMEM)],
        out_specs=pl.BlockSpec(memory_space=pltpu.VMEM),
        scratch_shapes=comm_bufs + [
            pltpu.SemaphoreType.DMA((NC,)),
            pltpu.SemaphoreType.DMA((NC,)),
            pltpu.SemaphoreType.DMA((NC,)),
            pltpu.SemaphoreType.DMA((NC,)),
        ],
        compiler_params=pltpu.CompilerParams(
            collective_id=0, vmem_limit_bytes=100 * 1024 * 1024),
    )(x, dy)
